# baseline (speedup 1.0000x reference)
import functools

import jax
import jax.numpy as jnp
from jax import lax
from jax.experimental import pallas as pl
from jax.experimental.pallas import tpu as pltpu
from jax.experimental.pallas import tpu_sc as plsc

N = 10000
D = 128
E = 320000

NC = 2
NS = 16
NW = NC * NS
CHUNK = 128
CPW = 80
G = 20
NG = CPW // G
E_PAD = NW * CPW * CHUNK
N_PAD = 10240
RPT = N_PAD // NS

_MESH = plsc.VectorSubcoreMesh(core_axis_name="c", subcore_axis_name="s")


def _sc_body(with_cnt, h_hbm, src_hbm, dst_hbm, agg_out, cnt_a, cnt_b,
             shared_agg, shared_cnt, rows, sg, dg, ones_v, zc_v,
             g0, g1, sis, sid_sem):
    cid = lax.axis_index("c")
    sid = lax.axis_index("s")
    wid = sid * NC + cid
    gsem = (g0, g1)

    pltpu.async_copy(src_hbm.at[wid * NG], sg[0], sis)
    pltpu.async_copy(dst_hbm.at[wid * NG], dg[0], sid_sem)

    zvec = jnp.zeros((16,), jnp.float32)

    def _zrow(i, _):
        for j in range(8):
            rows[0][i, pl.ds(j * 16, 16)] = zvec
        return 0

    lax.fori_loop(0, CHUNK, _zrow, 0)

    def _zc(i, _):
        zc_v[pl.ds(pl.multiple_of(i * 16, 16), 16)] = zvec
        return 0

    lax.fori_loop(0, RPT // 16, _zc, 0)

    if with_cnt:
        ovec = jnp.ones((16,), jnp.float32)

        def _ones(i, _):
            ones_v[pl.ds(pl.multiple_of(i * 16, 16), 16)] = ovec
            return 0

        lax.fori_loop(0, CHUNK // 16, _ones, 0)

    for u in range(RPT // CHUNK):
        pltpu.sync_copy(rows[0], shared_agg.at[pl.ds(sid * RPT + u * CHUNK, CHUNK)])
    pltpu.sync_copy(zc_v, shared_cnt.at[pl.ds(sid * RPT, RPT)])
    plsc.subcore_barrier()

    def _drain_idx(p):
        pltpu.make_async_copy(src_hbm.at[0], sg[p], sis).wait()
        pltpu.make_async_copy(dst_hbm.at[0], dg[p], sid_sem).wait()

    def _issue_gather(p, j, b):
        pltpu.async_copy(h_hbm.at[sg[p].at[j]], rows[b], gsem[b])

    def _drain_gather(b):
        pltpu.make_async_copy(h_hbm.at[pl.ds(0, CHUNK)], rows[b], gsem[b]).wait()

    def _scatter(p, j, b):
        pltpu.sync_copy(rows[b], shared_agg.at[dg[p].at[j]], add=True)
        if with_cnt:
            pltpu.sync_copy(ones_v, shared_cnt.at[dg[p].at[j]], add=True)

    for grp in range(NG):
        p = grp & 1
        _drain_idx(p)
        if grp + 1 < NG:
            pltpu.async_copy(src_hbm.at[wid * NG + (grp + 1)], sg[p ^ 1], sis)
            pltpu.async_copy(dst_hbm.at[wid * NG + (grp + 1)], dg[p ^ 1], sid_sem)
        _issue_gather(p, 0, 0)

        def _pair(g2, _):
            jb = g2 * 2
            _drain_gather(0)
            _issue_gather(p, jb + 1, 1)
            _scatter(p, jb, 0)
            _drain_gather(1)
            _issue_gather(p, jb + 2, 0)
            _scatter(p, jb + 1, 1)
            return 0

        lax.fori_loop(0, G // 2 - 1, _pair, 0)
        _drain_gather(0)
        _issue_gather(p, G - 1, 1)
        _scatter(p, G - 2, 0)
        _drain_gather(1)
        _scatter(p, G - 1, 1)
    plsc.subcore_barrier()

    row0 = sid * RPT
    pltpu.sync_copy(shared_agg.at[pl.ds(row0, RPT)],
                    agg_out.at[pl.ds(cid * N_PAD + row0, RPT)])
    if with_cnt:
        @pl.when(cid == 0)
        def _():
            pltpu.sync_copy(shared_cnt.at[pl.ds(row0, RPT)],
                            cnt_a.at[pl.ds(row0, RPT)])

        @pl.when(cid == 1)
        def _():
            pltpu.sync_copy(shared_cnt.at[pl.ds(row0, RPT)],
                            cnt_b.at[pl.ds(row0, RPT)])


def _make_sc(with_cnt):
    outs = [jax.ShapeDtypeStruct((NC * N_PAD, D), jnp.float32)]
    if with_cnt:
        outs += [jax.ShapeDtypeStruct((N_PAD,), jnp.float32)] * 2
    body = functools.partial(_sc_body, with_cnt)
    if not with_cnt:
        def body(h, s, dst, agg, *rest):
            return _sc_body(False, h, s, dst, agg, None, None, *rest)
    return pl.kernel(
        body,
        out_type=outs,
        mesh=_MESH,
        scratch_types=[
            pltpu.VMEM_SHARED((N_PAD, D), jnp.float32),
            pltpu.VMEM_SHARED((N_PAD,), jnp.float32),
            [pltpu.VMEM((CHUNK, D), jnp.float32) for _ in range(2)],
            [pltpu.VMEM((G, CHUNK), jnp.int32) for _ in range(2)],
            [pltpu.VMEM((G, CHUNK), jnp.int32) for _ in range(2)],
            pltpu.VMEM((CHUNK,), jnp.float32),
            pltpu.VMEM((RPT,), jnp.float32),
        ] + [pltpu.SemaphoreType.DMA] * 4,
    )


_sc_agg_cnt = _make_sc(True)
_sc_agg = _make_sc(False)

BN = 1024


def _tc_body(relu, agg0_ref, agg1_ref, ca_ref, cb_ref, h_ref, wn_ref, ws_ref,
             b_ref, out_ref):
    cnt = ca_ref[...] + cb_ref[...]
    inv = 1.0 / jnp.maximum(cnt, 1.0)
    agg = (agg0_ref[0] + agg1_ref[0]) * inv[:, None]
    acc = jnp.dot(agg, wn_ref[...], preferred_element_type=jnp.float32)
    acc += jnp.dot(h_ref[...], ws_ref[...], preferred_element_type=jnp.float32)
    acc += b_ref[...][None, :]
    if relu:
        acc = jnp.maximum(acc, 0.0)
    out_ref[...] = acc


def _make_tc(relu):
    grid = N_PAD // BN
    return pl.pallas_call(
        functools.partial(_tc_body, relu),
        grid=(grid,),
        in_specs=[
            pl.BlockSpec((1, BN, D), lambda i: (0, i, 0)),
            pl.BlockSpec((1, BN, D), lambda i: (1, i, 0)),
            pl.BlockSpec((BN,), lambda i: (i,)),
            pl.BlockSpec((BN,), lambda i: (i,)),
            pl.BlockSpec((BN, D), lambda i: (i, 0)),
            pl.BlockSpec((D, D), lambda i: (0, 0)),
            pl.BlockSpec((D, D), lambda i: (0, 0)),
            pl.BlockSpec((D,), lambda i: (0,)),
        ],
        out_specs=pl.BlockSpec((BN, D), lambda i: (i, 0)),
        out_shape=jax.ShapeDtypeStruct((N, D), jnp.float32),
    )


_tc_relu = _make_tc(True)
_tc_lin = _make_tc(False)


def kernel(x, edge_index, W_self_0, W_neigh_0, b_0, W_self_1, W_neigh_1, b_1,
           W_self_2, W_neigh_2, b_2):
    pad = E_PAD - E
    src = jnp.concatenate([edge_index[0], jnp.zeros((pad,), jnp.int32)])
    dst = jnp.concatenate([edge_index[1], jnp.full((pad,), N, jnp.int32)])
    src = src.reshape(NW * NG, G, CHUNK)
    dst = dst.reshape(NW * NG, G, CHUNK)

    agg_f, cnt_a, cnt_b = _sc_agg_cnt(x, src, dst)
    agg = agg_f.reshape(NC, N_PAD, D)
    h = _tc_relu(agg, agg, cnt_a, cnt_b, x, W_neigh_0, W_self_0, b_0)

    agg = _sc_agg(h, src, dst)[0].reshape(NC, N_PAD, D)
    h = _tc_relu(agg, agg, cnt_a, cnt_b, h, W_neigh_1, W_self_1, b_1)

    agg = _sc_agg(h, src, dst)[0].reshape(NC, N_PAD, D)
    return _tc_lin(agg, agg, cnt_a, cnt_b, h, W_neigh_2, W_self_2, b_2)


# speedup vs baseline: 1.0960x; 1.0960x over previous
import functools

import jax
import jax.numpy as jnp
from jax import lax
from jax.experimental import pallas as pl
from jax.experimental.pallas import tpu as pltpu
from jax.experimental.pallas import tpu_sc as plsc

N = 10000
D = 128
E = 320000

NC = 2
NS = 16
NW = NC * NS
CHUNK = 128
G = 20
CPT0 = 120
CPT1 = 40
NG0 = CPT0 // G
NG1 = CPT1 // G
TOTG = NS * (NG0 + NG1)
E_PAD = TOTG * G * CHUNK
N_PAD = 10240
RPT = N_PAD // NS

_MESH = plsc.VectorSubcoreMesh(core_axis_name="c", subcore_axis_name="s")


def _sc_body(with_cnt, h_hbm, src_hbm, dst_hbm, agg_out, cnt_a, cnt_b,
             shared_agg, shared_cnt, rows, sg, dg, ones_v, zc_v,
             g0, g1, sis, sid_sem):
    cid = lax.axis_index("c")
    sid = lax.axis_index("s")
    gsem = (g0, g1)

    my_ng = jnp.where(cid == 0, NG0, NG1)
    gbase = jnp.where(cid == 0, sid * NG0, NS * NG0 + sid * NG1)

    pltpu.async_copy(src_hbm.at[gbase], sg[0], sis)
    pltpu.async_copy(dst_hbm.at[gbase], dg[0], sid_sem)

    zvec = jnp.zeros((16,), jnp.float32)

    def _zrow(i, _):
        for j in range(8):
            rows[0][i, pl.ds(j * 16, 16)] = zvec
        return 0

    lax.fori_loop(0, CHUNK, _zrow, 0)

    def _zc(i, _):
        zc_v[pl.ds(pl.multiple_of(i * 16, 16), 16)] = zvec
        return 0

    lax.fori_loop(0, RPT // 16, _zc, 0)

    if with_cnt:
        ovec = jnp.ones((16,), jnp.float32)

        def _ones(i, _):
            ones_v[pl.ds(pl.multiple_of(i * 16, 16), 16)] = ovec
            return 0

        lax.fori_loop(0, CHUNK // 16, _ones, 0)

    for u in range(RPT // CHUNK):
        pltpu.sync_copy(rows[0], shared_agg.at[pl.ds(sid * RPT + u * CHUNK, CHUNK)])
    pltpu.sync_copy(zc_v, shared_cnt.at[pl.ds(sid * RPT, RPT)])
    plsc.subcore_barrier()

    def _drain_idx(p):
        pltpu.make_async_copy(src_hbm.at[0], sg[p], sis).wait()
        pltpu.make_async_copy(dst_hbm.at[0], dg[p], sid_sem).wait()

    def _issue_gather(p, j, b):
        pltpu.async_copy(h_hbm.at[sg[p].at[j]], rows[b], gsem[b])

    def _drain_gather(b):
        pltpu.make_async_copy(h_hbm.at[pl.ds(0, CHUNK)], rows[b], gsem[b]).wait()

    def _scatter(p, j, b):
        pltpu.sync_copy(rows[b], shared_agg.at[dg[p].at[j]], add=True)
        if with_cnt:
            pltpu.sync_copy(ones_v, shared_cnt.at[dg[p].at[j]], add=True)

    for grp in range(max(NG0, NG1)):
        p = grp & 1

        @pl.when(grp < my_ng)
        def _process():
            _drain_idx(p)

            @pl.when(grp + 1 < my_ng)
            def _stage_next():
                pltpu.async_copy(src_hbm.at[gbase + (grp + 1)], sg[p ^ 1], sis)
                pltpu.async_copy(dst_hbm.at[gbase + (grp + 1)], dg[p ^ 1],
                                 sid_sem)

            _issue_gather(p, 0, 0)

            def _pair(g2, _):
                jb = g2 * 2
                _drain_gather(0)
                _issue_gather(p, jb + 1, 1)
                _scatter(p, jb, 0)
                _drain_gather(1)
                _issue_gather(p, jb + 2, 0)
                _scatter(p, jb + 1, 1)
                return 0

            lax.fori_loop(0, G // 2 - 1, _pair, 0)
            _drain_gather(0)
            _issue_gather(p, G - 1, 1)
            _scatter(p, G - 2, 0)
            _drain_gather(1)
            _scatter(p, G - 1, 1)
    plsc.subcore_barrier()

    row0 = sid * RPT
    pltpu.sync_copy(shared_agg.at[pl.ds(row0, RPT)],
                    agg_out.at[pl.ds(cid * N_PAD + row0, RPT)])
    if with_cnt:
        @pl.when(cid == 0)
        def _():
            pltpu.sync_copy(shared_cnt.at[pl.ds(row0, RPT)],
                            cnt_a.at[pl.ds(row0, RPT)])

        @pl.when(cid == 1)
        def _():
            pltpu.sync_copy(shared_cnt.at[pl.ds(row0, RPT)],
                            cnt_b.at[pl.ds(row0, RPT)])


def _make_sc(with_cnt):
    outs = [jax.ShapeDtypeStruct((NC * N_PAD, D), jnp.float32)]
    if with_cnt:
        outs += [jax.ShapeDtypeStruct((N_PAD,), jnp.float32)] * 2
    body = functools.partial(_sc_body, with_cnt)
    if not with_cnt:
        def body(h, s, dst, agg, *rest):
            return _sc_body(False, h, s, dst, agg, None, None, *rest)
    return pl.kernel(
        body,
        out_type=outs,
        mesh=_MESH,
        scratch_types=[
            pltpu.VMEM_SHARED((N_PAD, D), jnp.float32),
            pltpu.VMEM_SHARED((N_PAD,), jnp.float32),
            [pltpu.VMEM((CHUNK, D), jnp.float32) for _ in range(2)],
            [pltpu.VMEM((G, CHUNK), jnp.int32) for _ in range(2)],
            [pltpu.VMEM((G, CHUNK), jnp.int32) for _ in range(2)],
            pltpu.VMEM((CHUNK,), jnp.float32),
            pltpu.VMEM((RPT,), jnp.float32),
        ] + [pltpu.SemaphoreType.DMA] * 4,
    )


_sc_agg_cnt = _make_sc(True)
_sc_agg = _make_sc(False)

BN = 1024


def _tc_body(relu, agg0_ref, agg1_ref, ca_ref, cb_ref, h_ref, wn_ref, ws_ref,
             b_ref, out_ref):
    cnt = ca_ref[...] + cb_ref[...]
    inv = 1.0 / jnp.maximum(cnt, 1.0)
    agg = (agg0_ref[0] + agg1_ref[0]) * inv[:, None]
    acc = jnp.dot(agg, wn_ref[...], preferred_element_type=jnp.float32)
    acc += jnp.dot(h_ref[...], ws_ref[...], preferred_element_type=jnp.float32)
    acc += b_ref[...][None, :]
    if relu:
        acc = jnp.maximum(acc, 0.0)
    out_ref[...] = acc


def _make_tc(relu):
    grid = N_PAD // BN
    return pl.pallas_call(
        functools.partial(_tc_body, relu),
        grid=(grid,),
        in_specs=[
            pl.BlockSpec((1, BN, D), lambda i: (0, i, 0)),
            pl.BlockSpec((1, BN, D), lambda i: (1, i, 0)),
            pl.BlockSpec((BN,), lambda i: (i,)),
            pl.BlockSpec((BN,), lambda i: (i,)),
            pl.BlockSpec((BN, D), lambda i: (i, 0)),
            pl.BlockSpec((D, D), lambda i: (0, 0)),
            pl.BlockSpec((D, D), lambda i: (0, 0)),
            pl.BlockSpec((D,), lambda i: (0,)),
        ],
        out_specs=pl.BlockSpec((BN, D), lambda i: (i, 0)),
        out_shape=jax.ShapeDtypeStruct((N, D), jnp.float32),
    )


_tc_relu = _make_tc(True)
_tc_lin = _make_tc(False)


def kernel(x, edge_index, W_self_0, W_neigh_0, b_0, W_self_1, W_neigh_1, b_1,
           W_self_2, W_neigh_2, b_2):
    pad = E_PAD - E
    src = jnp.concatenate([edge_index[0], jnp.zeros((pad,), jnp.int32)])
    dst = jnp.concatenate([edge_index[1], jnp.full((pad,), N, jnp.int32)])
    src = src.reshape(TOTG, G, CHUNK)
    dst = dst.reshape(TOTG, G, CHUNK)

    agg_f, cnt_a, cnt_b = _sc_agg_cnt(x, src, dst)
    agg = agg_f.reshape(NC, N_PAD, D)
    h = _tc_relu(agg, agg, cnt_a, cnt_b, x, W_neigh_0, W_self_0, b_0)

    agg = _sc_agg(h, src, dst)[0].reshape(NC, N_PAD, D)
    h = _tc_relu(agg, agg, cnt_a, cnt_b, h, W_neigh_1, W_self_1, b_1)

    agg = _sc_agg(h, src, dst)[0].reshape(NC, N_PAD, D)
    return _tc_lin(agg, agg, cnt_a, cnt_b, h, W_neigh_2, W_self_2, b_2)
